# baseline (speedup 1.0000x reference)
import jax
import jax.numpy as jnp
from jax.experimental import pallas as pl


_ROWS = 16384
_COLS = 4096
_BLK = 512


def _copy_body(in_ref, out_ref):
    out_ref[...] = in_ref[...]


def kernel(tensor):
    flat = tensor.reshape(_ROWS, _COLS)
    out = pl.pallas_call(
        _copy_body,
        grid=(_ROWS // _BLK,),
        in_specs=[pl.BlockSpec((_BLK, _COLS), lambda i: (i, 0))],
        out_specs=pl.BlockSpec((_BLK, _COLS), lambda i: (i, 0)),
        out_shape=jax.ShapeDtypeStruct((_ROWS, _COLS), jnp.float32),
    )(flat)
    return out.reshape(tensor.shape[0], tensor.shape[1], _COLS)
